# baseline (speedup 1.0000x reference)
import functools

import jax
import jax.numpy as jnp
from jax import lax
from jax.experimental import pallas as pl
from jax.experimental.pallas import tpu as pltpu
from jax.experimental.pallas import tpu_sc as plsc

E = 160000
D = 256
N_DST = 10000


BM = 1280


def _mmT_body(w_ref, x_ref, b_ref, o_ref):
    acc = lax.dot_general(w_ref[...], x_ref[...], (((1,), (1,)), ((), ())),
                          preferred_element_type=jnp.float32)
    o_ref[...] = jnp.maximum(acc + b_ref[...], 0.0)


def _edge_linear_T(src_emb, W, b):
    return pl.pallas_call(
        _mmT_body,
        grid=(E // BM,),
        in_specs=[
            pl.BlockSpec((D, D), lambda i: (0, 0)),
            pl.BlockSpec((BM, D), lambda i: (i, 0)),
            pl.BlockSpec((D, 1), lambda i: (0, 0)),
        ],
        out_specs=pl.BlockSpec((D, BM), lambda i: (0, i)),
        out_shape=jax.ShapeDtypeStruct((D, E), jnp.float32),
    )(W, src_emb, b.reshape(D, 1))


NC = 2
NS = 16
NW = NC * NS
CPT = D // NW
CHUNK = 3200
NCHUNKS = E // CHUNK


def _sc_body(blk_hbm, msgT_hbm, out_hbm, idx_v, val_v, acc_v):
    wid = lax.axis_index("s") * NC + lax.axis_index("c")
    row0 = wid * CPT

    zero16 = jnp.zeros((16,), jnp.float32)

    def zero_body(i, _):
        for c in range(CPT):
            acc_v[c, pl.ds(i * 16, 16)] = zero16
        return 0

    lax.fori_loop(0, N_DST // 16, zero_body, 0)

    def chunk_body(k, _):
        pltpu.sync_copy(blk_hbm.at[pl.ds(k * CHUNK, CHUNK)], idx_v)
        pltpu.sync_copy(
            msgT_hbm.at[pl.ds(row0, CPT), pl.ds(k * CHUNK, CHUNK)], val_v)

        def step(j, _):
            d16 = idx_v[pl.ds(j * 16, 16)]
            cidx = [jnp.full((16,), c, jnp.int32) for c in range(CPT)]
            vals = []
            pend_any = None
            for c in range(CPT):
                v = val_v[c, pl.ds(j * 16, 16)]
                a = plsc.load_gather(acc_v, [cidx[c], d16])
                m = jnp.maximum(a, v)
                plsc.store_scatter(acc_v, [cidx[c], d16], m)
                got = plsc.load_gather(acc_v, [cidx[c], d16])
                p = got < m
                pend_any = p if pend_any is None else jnp.logical_or(pend_any, p)
                vals.append(m)

            @pl.when(jnp.any(pend_any))
            def _fix():
                for c in range(CPT):
                    def cond(carry):
                        return jnp.any(carry[0])

                    def body(carry):
                        pend, m = carry
                        a = plsc.load_gather(acc_v, [cidx[c], d16])
                        m2 = jnp.maximum(a, m)
                        plsc.store_scatter(acc_v, [cidx[c], d16], m2,
                                           mask=pend)
                        got = plsc.load_gather(acc_v, [cidx[c], d16])
                        return jnp.logical_and(pend, got < m2), m2

                    lax.while_loop(
                        cond, body,
                        (jnp.ones((16,), jnp.bool_), vals[c]))

            return 0

        lax.fori_loop(0, CHUNK // 16, step, 0)
        return 0

    lax.fori_loop(0, NCHUNKS, chunk_body, 0)

    pltpu.sync_copy(acc_v, out_hbm.at[wid])


@functools.partial(
    pl.kernel,
    out_type=jax.ShapeDtypeStruct((NW, CPT, N_DST), jnp.float32),
    mesh=plsc.VectorSubcoreMesh(core_axis_name="c", subcore_axis_name="s"),
    scratch_types=[
        pltpu.VMEM((CHUNK,), jnp.int32),
        pltpu.VMEM((CPT, CHUNK), jnp.float32),
        pltpu.VMEM((CPT, N_DST), jnp.float32),
    ],
    compiler_params=pltpu.CompilerParams(needs_layout_passes=False),
)
def _sc_scatter_max(blk_hbm, msgT_hbm, out_hbm, idx_v, val_v, acc_v):
    _sc_body(blk_hbm, msgT_hbm, out_hbm, idx_v, val_v, acc_v)


def kernel(block, src_emb, src_emb_in, W, b):
    msgT = _edge_linear_T(src_emb, W, b)
    hT = _sc_scatter_max(block.astype(jnp.int32), msgT).reshape(D, N_DST)
    return hT.T + src_emb[E:, :]


# speedup vs baseline: 1.0393x; 1.0393x over previous
import functools

import jax
import jax.numpy as jnp
from jax import lax
from jax.experimental import pallas as pl
from jax.experimental.pallas import tpu as pltpu
from jax.experimental.pallas import tpu_sc as plsc

E = 160000
D = 256
N_DST = 10000


BM = 1280


def _mmT_body(w_ref, x_ref, b_ref, o_ref):
    acc = lax.dot_general(w_ref[...], x_ref[...], (((1,), (1,)), ((), ())),
                          preferred_element_type=jnp.float32)
    o_ref[...] = jnp.maximum(acc + b_ref[...], 0.0)


def _edge_linear_T(src_emb, W, b):
    return pl.pallas_call(
        _mmT_body,
        grid=(E // BM,),
        in_specs=[
            pl.BlockSpec((D, D), lambda i: (0, 0)),
            pl.BlockSpec((BM, D), lambda i: (i, 0)),
            pl.BlockSpec((D, 1), lambda i: (0, 0)),
        ],
        out_specs=pl.BlockSpec((D, BM), lambda i: (0, i)),
        out_shape=jax.ShapeDtypeStruct((D, E), jnp.float32),
    )(W, src_emb, b.reshape(D, 1))


NC = 2
NS = 16
NW = NC * NS
CPT = D // NW
CHUNK = 3200
NCHUNKS = E // CHUNK


def _sc_body(blk_hbm, msgT_hbm, out_hbm, idx_v, val_v, acc_v, dup_v):
    wid = lax.axis_index("s") * NC + lax.axis_index("c")
    row0 = wid * CPT

    zero16 = jnp.zeros((16,), jnp.float32)
    iota16 = lax.iota(jnp.int32, 16)

    def zero_body(i, _):
        acc_v[pl.ds(i * 16, 16)] = zero16
        return 0

    lax.fori_loop(0, (CPT * N_DST) // 16, zero_body, 0)

    def chunk_body(k, _):
        pltpu.sync_copy(blk_hbm.at[pl.ds(k * CHUNK, CHUNK)], idx_v)
        pltpu.sync_copy(
            msgT_hbm.at[pl.ds(row0, CPT), pl.ds(k * CHUNK, CHUNK)], val_v)

        def step(j, _):
            d16 = idx_v[pl.ds(j * 16, 16)]
            plsc.store_scatter(dup_v, [d16], iota16)
            got = plsc.load_gather(dup_v, [d16])
            no_dup = jnp.all(got == iota16)

            @pl.when(no_dup)
            def _fast():
                for c in range(CPT):
                    fidx = d16 + (c * N_DST)
                    v = val_v[c, pl.ds(j * 16, 16)]
                    a = plsc.load_gather(acc_v, [fidx])
                    plsc.store_scatter(acc_v, [fidx], jnp.maximum(a, v))

            @pl.when(jnp.logical_not(no_dup))
            def _slow():
                for c in range(CPT):
                    fidx = d16 + (c * N_DST)
                    v = val_v[c, pl.ds(j * 16, 16)]

                    def cond(carry):
                        return jnp.any(carry[0])

                    def body(carry):
                        pend, m = carry
                        a = plsc.load_gather(acc_v, [fidx])
                        m2 = jnp.maximum(a, m)
                        plsc.store_scatter(acc_v, [fidx], m2, mask=pend)
                        g2 = plsc.load_gather(acc_v, [fidx])
                        return jnp.logical_and(pend, g2 < m2), m2

                    lax.while_loop(cond, body,
                                   (jnp.ones((16,), jnp.bool_), v))

            return 0

        lax.fori_loop(0, CHUNK // 16, step, 0)
        return 0

    lax.fori_loop(0, NCHUNKS, chunk_body, 0)

    pltpu.sync_copy(acc_v, out_hbm.at[wid])


@functools.partial(
    pl.kernel,
    out_type=jax.ShapeDtypeStruct((NW, CPT * N_DST), jnp.float32),
    mesh=plsc.VectorSubcoreMesh(core_axis_name="c", subcore_axis_name="s"),
    scratch_types=[
        pltpu.VMEM((CHUNK,), jnp.int32),
        pltpu.VMEM((CPT, CHUNK), jnp.float32),
        pltpu.VMEM((CPT * N_DST,), jnp.float32),
        pltpu.VMEM((N_DST,), jnp.int32),
    ],
    compiler_params=pltpu.CompilerParams(needs_layout_passes=False),
)
def _sc_scatter_max(blk_hbm, msgT_hbm, out_hbm, idx_v, val_v, acc_v, dup_v):
    _sc_body(blk_hbm, msgT_hbm, out_hbm, idx_v, val_v, acc_v, dup_v)


def kernel(block, src_emb, src_emb_in, W, b):
    msgT = _edge_linear_T(src_emb, W, b)
    hT = _sc_scatter_max(block.astype(jnp.int32), msgT).reshape(D, N_DST)
    return hT.T + src_emb[E:, :]


# speedup vs baseline: 1.1359x; 1.0929x over previous
import functools

import jax
import jax.numpy as jnp
from jax import lax
from jax.experimental import pallas as pl
from jax.experimental.pallas import tpu as pltpu
from jax.experimental.pallas import tpu_sc as plsc

E = 160000
D = 256
N_DST = 10000
PAD_N = 10240


BM = 1280


def _mmT_body(w_ref, x_ref, b_ref, o_ref):
    acc = lax.dot_general(w_ref[...], x_ref[...], (((1,), (1,)), ((), ())),
                          preferred_element_type=jnp.float32)
    o_ref[...] = jnp.maximum(acc + b_ref[...], 0.0)


def _edge_linear_T(src_emb, W, b):
    return pl.pallas_call(
        _mmT_body,
        grid=(E // BM,),
        in_specs=[
            pl.BlockSpec((D, D), lambda i: (0, 0)),
            pl.BlockSpec((BM, D), lambda i: (i, 0)),
            pl.BlockSpec((D, 1), lambda i: (0, 0)),
        ],
        out_specs=pl.BlockSpec((D, BM), lambda i: (0, i)),
        out_shape=jax.ShapeDtypeStruct((D, E), jnp.float32),
    )(W, src_emb, b.reshape(D, 1))


NC = 2
NS = 16
NW = NC * NS
CPT = D // NW
CHUNK = 3200
NCHUNKS = E // CHUNK
NSTEP = CHUNK // 16


def _sc_body(blk_hbm, msgT_hbm, out_hbm, idx_v, val_v, flag_v, *accs):
    wid = lax.axis_index("s") * NC + lax.axis_index("c")
    row0 = wid * CPT

    zero16 = jnp.zeros((16,), jnp.float32)

    def zero_body(i, _):
        for c in range(CPT):
            accs[c][pl.ds(i * 16, 16)] = zero16
        return 0

    lax.fori_loop(0, PAD_N // 16, zero_body, 0)

    def chunk_body(k, _):
        pltpu.sync_copy(blk_hbm.at[pl.ds(k * CHUNK, CHUNK)], idx_v)
        pltpu.sync_copy(
            msgT_hbm.at[pl.ds(row0, CPT), pl.ds(k * CHUNK, CHUNK)], val_v)

        def flag_body(s, _):
            d16 = idx_v[pl.ds(s * 16, 16)]
            _, last = plsc.scan_count(d16)
            flag_v[pl.ds(s * 16, 16)] = plsc.all_reduce_population_count(last)
            return 0

        lax.fori_loop(0, NSTEP, flag_body, 0)

        def step(j, _):
            d16 = idx_v[pl.ds(j * 16, 16)]
            nodup = flag_v[pl.ds(j * 16, 16)][0] == 16

            @pl.when(nodup)
            def _fast():
                for c in range(CPT):
                    v = val_v[c, pl.ds(j * 16, 16)]
                    a = plsc.load_gather(accs[c], [d16])
                    plsc.store_scatter(accs[c], [d16], jnp.maximum(a, v))

            @pl.when(jnp.logical_not(nodup))
            def _slow():
                for c in range(CPT):
                    v = val_v[c, pl.ds(j * 16, 16)]

                    def cond(carry):
                        return jnp.any(carry[0])

                    def body(carry):
                        pend, m = carry
                        a = plsc.load_gather(accs[c], [d16])
                        m2 = jnp.maximum(a, m)
                        plsc.store_scatter(accs[c], [d16], m2, mask=pend)
                        g2 = plsc.load_gather(accs[c], [d16])
                        return jnp.logical_and(pend, g2 < m2), m2

                    lax.while_loop(cond, body,
                                   (jnp.ones((16,), jnp.bool_), v))

            return 0

        lax.fori_loop(0, NSTEP, step, 0)
        return 0

    lax.fori_loop(0, NCHUNKS, chunk_body, 0)

    for c in range(CPT):
        pltpu.sync_copy(accs[c], out_hbm.at[row0 + c])


@functools.partial(
    pl.kernel,
    out_type=jax.ShapeDtypeStruct((D, PAD_N), jnp.float32),
    mesh=plsc.VectorSubcoreMesh(core_axis_name="c", subcore_axis_name="s"),
    scratch_types=[
        pltpu.VMEM((CHUNK,), jnp.int32),
        pltpu.VMEM((CPT, CHUNK), jnp.float32),
        pltpu.VMEM((CHUNK,), jnp.int32),
    ] + [pltpu.VMEM((PAD_N,), jnp.float32) for _ in range(CPT)],
    compiler_params=pltpu.CompilerParams(needs_layout_passes=False),
)
def _sc_scatter_max(blk_hbm, msgT_hbm, out_hbm, idx_v, val_v, flag_v, *accs):
    _sc_body(blk_hbm, msgT_hbm, out_hbm, idx_v, val_v, flag_v, *accs)


def kernel(block, src_emb, src_emb_in, W, b):
    msgT = _edge_linear_T(src_emb, W, b)
    hT = _sc_scatter_max(block.astype(jnp.int32), msgT)
    return hT[:, :N_DST].T + src_emb[E:, :]


# speedup vs baseline: 1.8856x; 1.6600x over previous
import functools

import jax
import jax.numpy as jnp
from jax import lax
from jax.experimental import pallas as pl
from jax.experimental.pallas import tpu as pltpu
from jax.experimental.pallas import tpu_sc as plsc

E = 160000
D = 256
HD = D // 2
N_DST = 10000
PAD_N = 10240


BM = 1280


def _mmT_body(w_ref, x_ref, b_ref, o_ref):
    acc = lax.dot_general(w_ref[...], x_ref[...], (((1,), (1,)), ((), ())),
                          preferred_element_type=jnp.float32)
    msg = jnp.maximum(acc + b_ref[...], 0.0)
    top = lax.bitcast_convert_type(
        msg[:HD, :].astype(jnp.bfloat16), jnp.uint16).astype(jnp.uint32)
    bot = lax.bitcast_convert_type(
        msg[HD:, :].astype(jnp.bfloat16), jnp.uint16).astype(jnp.uint32)
    o_ref[...] = (top | (bot << 16)).astype(jnp.int32)


def _edge_linear_packed(src_emb, W, b):
    return pl.pallas_call(
        _mmT_body,
        grid=(E // BM,),
        in_specs=[
            pl.BlockSpec((D, D), lambda i: (0, 0)),
            pl.BlockSpec((BM, D), lambda i: (i, 0)),
            pl.BlockSpec((D, 1), lambda i: (0, 0)),
        ],
        out_specs=pl.BlockSpec((HD, BM), lambda i: (0, i)),
        out_shape=jax.ShapeDtypeStruct((HD, E), jnp.int32),
    )(W, src_emb, b.reshape(D, 1))


NC = 2
NS = 16
RPT = HD // NS
EH = E // NC
CHUNK = 3200
NCHUNKS = EH // CHUNK
NSTEP = CHUNK // 16


def _bmax(a, b):
    return plsc.bitcast(
        jnp.maximum(plsc.bitcast(a, jnp.bfloat16),
                    plsc.bitcast(b, jnp.bfloat16)), jnp.int32)


def _sc_body(blk_hbm, msgP_hbm, out_hbm, idx_v, val_v, flag_v, *accs):
    sc = lax.axis_index("c")
    sid = lax.axis_index("s")
    row0 = sid * RPT
    e0 = sc * EH

    zero16 = jnp.zeros((16,), jnp.int32)

    def zero_body(i, _):
        for c in range(RPT):
            accs[c][pl.ds(i * 16, 16)] = zero16
        return 0

    lax.fori_loop(0, PAD_N // 16, zero_body, 0)

    def chunk_body(k, _):
        pltpu.sync_copy(blk_hbm.at[pl.ds(e0 + k * CHUNK, CHUNK)], idx_v)
        pltpu.sync_copy(
            msgP_hbm.at[pl.ds(row0, RPT), pl.ds(e0 + k * CHUNK, CHUNK)],
            val_v)

        def flag_body(s, _):
            d16 = idx_v[pl.ds(s * 16, 16)]
            _, last = plsc.scan_count(d16)
            flag_v[pl.ds(s * 16, 16)] = plsc.all_reduce_population_count(last)
            return 0

        lax.fori_loop(0, NSTEP, flag_body, 0)

        def step(j, _):
            d16 = idx_v[pl.ds(j * 16, 16)]
            nodup = flag_v[pl.ds(j * 16, 16)][0] == 16

            @pl.when(nodup)
            def _fast():
                for c in range(RPT):
                    v = val_v[c, pl.ds(j * 16, 16)]
                    a = plsc.load_gather(accs[c], [d16])
                    plsc.store_scatter(accs[c], [d16], _bmax(a, v))

            @pl.when(jnp.logical_not(nodup))
            def _slow():
                for c in range(RPT):
                    v = val_v[c, pl.ds(j * 16, 16)]

                    def cond(carry):
                        return jnp.any(carry[0])

                    def body(carry):
                        pend, m = carry
                        a = plsc.load_gather(accs[c], [d16])
                        m2 = _bmax(a, m)
                        plsc.store_scatter(accs[c], [d16], m2, mask=pend)
                        g2 = plsc.load_gather(accs[c], [d16])
                        return jnp.logical_and(pend, g2 != m2), m2

                    lax.while_loop(cond, body,
                                   (jnp.ones((16,), jnp.bool_), v))

            return 0

        lax.fori_loop(0, NSTEP, step, 0)
        return 0

    lax.fori_loop(0, NCHUNKS, chunk_body, 0)

    for c in range(RPT):
        pltpu.sync_copy(accs[c], out_hbm.at[sc, row0 + c])


@functools.partial(
    pl.kernel,
    out_type=jax.ShapeDtypeStruct((NC, HD, PAD_N), jnp.int32),
    mesh=plsc.VectorSubcoreMesh(core_axis_name="c", subcore_axis_name="s"),
    scratch_types=[
        pltpu.VMEM((CHUNK,), jnp.int32),
        pltpu.VMEM((RPT, CHUNK), jnp.int32),
        pltpu.VMEM((CHUNK,), jnp.int32),
    ] + [pltpu.VMEM((PAD_N,), jnp.int32) for _ in range(RPT)],
    compiler_params=pltpu.CompilerParams(needs_layout_passes=False),
)
def _sc_scatter_max(blk_hbm, msgP_hbm, out_hbm, idx_v, val_v, flag_v, *accs):
    _sc_body(blk_hbm, msgP_hbm, out_hbm, idx_v, val_v, flag_v, *accs)


def _unpack(p):
    lo = lax.bitcast_convert_type(
        (p & 0xFFFF).astype(jnp.uint16), jnp.bfloat16)
    hi = lax.bitcast_convert_type(
        ((p >> 16) & 0xFFFF).astype(jnp.uint16), jnp.bfloat16)
    return lo, hi


def kernel(block, src_emb, src_emb_in, W, b):
    msgP = _edge_linear_packed(src_emb, W, b)
    parts = _sc_scatter_max(block.astype(jnp.int32), msgP)
    lo0, hi0 = _unpack(parts[0])
    lo1, hi1 = _unpack(parts[1])
    top = jnp.maximum(lo0, lo1).astype(jnp.float32)
    bot = jnp.maximum(hi0, hi1).astype(jnp.float32)
    hT = jnp.concatenate([top, bot], axis=0)
    return hT[:, :N_DST].T + src_emb[E:, :]


# speedup vs baseline: 2.0746x; 1.1003x over previous
import functools

import jax
import jax.numpy as jnp
from jax import lax
from jax.experimental import pallas as pl
from jax.experimental.pallas import tpu as pltpu
from jax.experimental.pallas import tpu_sc as plsc

E = 160000
D = 256
HD = D // 2
N_DST = 10000
PAD_N = 10240


BM = 1280


def _mmT_body(w_ref, x_ref, b_ref, o_ref):
    acc = lax.dot_general(w_ref[...], x_ref[...], (((1,), (1,)), ((), ())),
                          preferred_element_type=jnp.float32)
    msg = jnp.maximum(acc + b_ref[...], 0.0)
    top = lax.bitcast_convert_type(
        msg[:HD, :].astype(jnp.bfloat16), jnp.uint16).astype(jnp.uint32)
    bot = lax.bitcast_convert_type(
        msg[HD:, :].astype(jnp.bfloat16), jnp.uint16).astype(jnp.uint32)
    o_ref[...] = (top | (bot << 16)).astype(jnp.int32)


def _edge_linear_packed(src_emb, W, b):
    return pl.pallas_call(
        _mmT_body,
        grid=(E // BM,),
        in_specs=[
            pl.BlockSpec((D, D), lambda i: (0, 0)),
            pl.BlockSpec((BM, D), lambda i: (i, 0)),
            pl.BlockSpec((D, 1), lambda i: (0, 0)),
        ],
        out_specs=pl.BlockSpec((HD, BM), lambda i: (0, i)),
        out_shape=jax.ShapeDtypeStruct((HD, E), jnp.int32),
    )(W, src_emb, b.reshape(D, 1))


NC = 2
NS = 16
RPT = HD // NS
EH = E // NC
CHUNK = 640
NCHUNKS = EH // CHUNK
NSTEP = CHUNK // 16


def _bmax(a, b):
    return plsc.bitcast(
        jnp.maximum(plsc.bitcast(a, jnp.bfloat16),
                    plsc.bitcast(b, jnp.bfloat16)), jnp.int32)


def _sc_body(blk_hbm, msgP_hbm, out_hbm, idx0, idx1, val0, val1, flag_v,
             sem_i0, sem_i1, sem_v0, sem_v1, *accs):
    sc = lax.axis_index("c")
    sid = lax.axis_index("s")
    row0 = sid * RPT
    e0 = sc * EH

    idxs = (idx0, idx1)
    vals = (val0, val1)
    sems_i = (sem_i0, sem_i1)
    sems_v = (sem_v0, sem_v1)

    zero16 = jnp.zeros((16,), jnp.int32)

    def zero_body(i, _):
        for c in range(RPT):
            accs[c][pl.ds(i * 16, 16)] = zero16
        return 0

    lax.fori_loop(0, PAD_N // 16, zero_body, 0)

    def issue(k, b):
        pltpu.async_copy(blk_hbm.at[pl.ds(e0 + k * CHUNK, CHUNK)],
                         idxs[b], sems_i[b])
        pltpu.async_copy(
            msgP_hbm.at[pl.ds(row0, RPT), pl.ds(e0 + k * CHUNK, CHUNK)],
            vals[b], sems_v[b])

    def wait(b):
        pltpu.make_async_copy(blk_hbm.at[pl.ds(e0, CHUNK)],
                              idxs[b], sems_i[b]).wait()
        pltpu.make_async_copy(
            msgP_hbm.at[pl.ds(row0, RPT), pl.ds(e0, CHUNK)],
            vals[b], sems_v[b]).wait()

    def process(k, b):
        idx_v = idxs[b]
        val_v = vals[b]

        def flag_body(s, _):
            d16 = idx_v[pl.ds(s * 16, 16)]
            _, last = plsc.scan_count(d16)
            flag_v[pl.ds(s * 16, 16)] = plsc.all_reduce_population_count(last)
            return 0

        lax.fori_loop(0, NSTEP, flag_body, 0)

        def step(j, _):
            d16 = idx_v[pl.ds(j * 16, 16)]
            nodup = flag_v[pl.ds(j * 16, 16)][0] == 16

            @pl.when(nodup)
            def _fast():
                for c in range(RPT):
                    v = val_v[c, pl.ds(j * 16, 16)]
                    a = plsc.load_gather(accs[c], [d16])
                    plsc.store_scatter(accs[c], [d16], _bmax(a, v))

            @pl.when(jnp.logical_not(nodup))
            def _slow():
                for c in range(RPT):
                    v = val_v[c, pl.ds(j * 16, 16)]

                    def cond(carry):
                        return jnp.any(carry[0])

                    def body(carry):
                        pend, m = carry
                        a = plsc.load_gather(accs[c], [d16])
                        m2 = _bmax(a, m)
                        plsc.store_scatter(accs[c], [d16], m2, mask=pend)
                        g2 = plsc.load_gather(accs[c], [d16])
                        return jnp.logical_and(pend, g2 != m2), m2

                    lax.while_loop(cond, body,
                                   (jnp.ones((16,), jnp.bool_), v))

            return 0

        lax.fori_loop(0, NSTEP, step, 0)

    issue(0, 0)

    def pair_body(t, _):
        k0 = t * 2

        wait(0)
        pl.when(k0 + 1 < NCHUNKS)(lambda: issue(k0 + 1, 1))
        process(k0, 0)

        @pl.when(k0 + 1 < NCHUNKS)
        def _odd():
            wait(1)
            pl.when(k0 + 2 < NCHUNKS)(lambda: issue(k0 + 2, 0))
            process(k0 + 1, 1)

        return 0

    lax.fori_loop(0, (NCHUNKS + 1) // 2, pair_body, 0)

    for c in range(RPT):
        pltpu.sync_copy(accs[c], out_hbm.at[sc, row0 + c])


@functools.partial(
    pl.kernel,
    out_type=jax.ShapeDtypeStruct((NC, HD, PAD_N), jnp.int32),
    mesh=plsc.VectorSubcoreMesh(core_axis_name="c", subcore_axis_name="s"),
    scratch_types=[
        pltpu.VMEM((CHUNK,), jnp.int32),
        pltpu.VMEM((CHUNK,), jnp.int32),
        pltpu.VMEM((RPT, CHUNK), jnp.int32),
        pltpu.VMEM((RPT, CHUNK), jnp.int32),
        pltpu.VMEM((CHUNK,), jnp.int32),
        pltpu.SemaphoreType.DMA,
        pltpu.SemaphoreType.DMA,
        pltpu.SemaphoreType.DMA,
        pltpu.SemaphoreType.DMA,
    ] + [pltpu.VMEM((PAD_N,), jnp.int32) for _ in range(RPT)],
    compiler_params=pltpu.CompilerParams(needs_layout_passes=False),
)
def _sc_scatter_max(blk_hbm, msgP_hbm, out_hbm, *scratch):
    _sc_body(blk_hbm, msgP_hbm, out_hbm, *scratch)


def _unpack(p):
    lo = lax.bitcast_convert_type(
        (p & 0xFFFF).astype(jnp.uint16), jnp.bfloat16)
    hi = lax.bitcast_convert_type(
        ((p >> 16) & 0xFFFF).astype(jnp.uint16), jnp.bfloat16)
    return lo, hi


def kernel(block, src_emb, src_emb_in, W, b):
    msgP = _edge_linear_packed(src_emb, W, b)
    parts = _sc_scatter_max(block.astype(jnp.int32), msgP)
    lo0, hi0 = _unpack(parts[0])
    lo1, hi1 = _unpack(parts[1])
    top = jnp.maximum(lo0, lo1).astype(jnp.float32)
    bot = jnp.maximum(hi0, hi1).astype(jnp.float32)
    hT = jnp.concatenate([top, bot], axis=0)
    return hT[:, :N_DST].T + src_emb[E:, :]
